# baseline (speedup 1.0000x reference)
import functools

import jax
import jax.numpy as jnp
import numpy as np
from jax import lax
from jax.experimental import pallas as pl
from jax.experimental.pallas import tpu as pltpu
from jax.experimental.pallas import tpu_sc as plsc

NUM_EMB = 105220
D = 256
G = 128
L = 50
N = 6400
H = 128

NC, NS = 2, 16
NW = NC * NS
CHUNKS = (128, 72)
LANES = 16


def _sc_gather(table, idx, d, clamp_max=None):
    n_rows = idx.shape[0]
    rpw = n_rows // NW
    assert rpw == sum(CHUNKS)
    n_vecs = -(-rpw // LANES)
    pad = n_vecs * LANES
    mesh = plsc.VectorSubcoreMesh(core_axis_name="c", subcore_axis_name="s")

    @functools.partial(
        pl.kernel,
        out_type=jax.ShapeDtypeStruct((n_rows, d), jnp.float32),
        mesh=mesh,
        scratch_types=[
            pltpu.VMEM((pad,), jnp.int32),
            pltpu.VMEM((rpw, d), jnp.float32),
            pltpu.SemaphoreType.DMA,
            pltpu.SemaphoreType.DMA,
        ],
    )
    def gather_kernel(idx_hbm, table_hbm, out_hbm, idx_v, rows_v, sem_g, sem_w):
        wid = lax.axis_index("s") * NC + lax.axis_index("c")
        base = wid * rpw
        pltpu.sync_copy(idx_hbm.at[pl.ds(base, rpw)], idx_v.at[pl.ds(0, rpw)])
        if clamp_max is not None:
            hi = jnp.full((LANES,), clamp_max, jnp.int32)
            lo = jnp.zeros((LANES,), jnp.int32)
            for v in range(n_vecs):
                sl = pl.ds(v * LANES, LANES)
                idx_v[sl] = jnp.minimum(jnp.maximum(idx_v[sl], lo), hi)
        off = 0
        gathers = []
        for c in CHUNKS:
            gathers.append(
                pltpu.async_copy(
                    table_hbm.at[idx_v.at[pl.ds(off, c)]],
                    rows_v.at[pl.ds(off, c)],
                    sem_g,
                )
            )
            off += c
        writes = []
        off = 0
        for g, c in zip(gathers, CHUNKS):
            g.wait()
            writes.append(
                pltpu.async_copy(
                    rows_v.at[pl.ds(off, c)],
                    out_hbm.at[pl.ds(base + off, c)],
                    sem_w,
                )
            )
            off += c
        for w in writes:
            w.wait()

    return gather_kernel(idx, table)


NSTRIPE = 8
SROWS = G * L // NSTRIPE


def _project_body(e_hbm, w1_ref, b1_ref, w2_ref, b2_ref, batch_ref, x_ref,
                  e_v, *sems):
    copies = [
        pltpu.make_async_copy(
            e_hbm.at[pl.ds(k * SROWS, SROWS)],
            e_v.at[pl.ds(k * SROWS, SROWS)],
            sems[k],
        )
        for k in range(NSTRIPE)
    ]
    for c in copies:
        c.start()

    gps = G // NSTRIPE
    row = lax.broadcasted_iota(jnp.int32, (gps, SROWS), 0)
    col = lax.broadcasted_iota(jnp.int32, (gps, SROWS), 1)
    off = col - row * L
    seg = jnp.where((off >= 0) & (off < L), 1.0 / L, 0.0)

    hms = []
    for k in range(NSTRIPE):
        copies[k].wait()
        e = e_v[pl.ds(k * SROWS, SROWS), :]
        h = jnp.dot(e, w1_ref[...], preferred_element_type=jnp.float32)
        h = h + b1_ref[...]
        h = jnp.where(h >= 0, h, 0.01 * h)
        hms.append(jnp.dot(seg, h, preferred_element_type=jnp.float32))

    hm = jnp.concatenate(hms, axis=0)
    m = jnp.dot(hm, w2_ref[...], preferred_element_type=jnp.float32)
    m = m + b2_ref[...]
    out_copies = []
    for k in range(NSTRIPE):
        sl = pl.ds(k * SROWS, SROWS)
        b_row = jnp.broadcast_to(batch_ref[:, sl], (G, SROWS))
        g_col = lax.broadcasted_iota(jnp.int32, (G, SROWS), 0)
        oh_t = jnp.where(b_row == g_col, 1.0, 0.0)
        e_v[sl, :] = lax.dot_general(
            oh_t, m, (((0,), (0,)), ((), ())),
            preferred_element_type=jnp.float32)
        cp = pltpu.make_async_copy(e_v.at[sl], x_ref.at[sl], sems[k])
        cp.start()
        out_copies.append(cp)
    for cp in out_copies:
        cp.wait()


def _project(e, W1, b1, W2, b2, batch):
    return pl.pallas_call(
        _project_body,
        in_specs=[
            pl.BlockSpec(memory_space=pl.ANY),
            pl.BlockSpec(memory_space=pltpu.MemorySpace.VMEM),
            pl.BlockSpec(memory_space=pltpu.MemorySpace.VMEM),
            pl.BlockSpec(memory_space=pltpu.MemorySpace.VMEM),
            pl.BlockSpec(memory_space=pltpu.MemorySpace.VMEM),
            pl.BlockSpec(memory_space=pltpu.MemorySpace.VMEM),
        ],
        out_specs=pl.BlockSpec(memory_space=pl.ANY),
        out_shape=jax.ShapeDtypeStruct((N, D), jnp.float32),
        scratch_shapes=(
            [pltpu.VMEM((G * L, D), jnp.float32)]
            + [pltpu.SemaphoreType.DMA] * NSTRIPE
        ),
    )(e, W1, b1.reshape(1, H), W2, b2.reshape(1, D), batch.reshape(1, N))


def kernel(original_ids, batch, emb_table, W1, b1, W2, b2):
    ids = original_ids.astype(jnp.int32).reshape(-1)
    e = _sc_gather(emb_table, ids, D, clamp_max=NUM_EMB - 1)
    return _project(e, W1, b1, W2, b2, batch.astype(jnp.int32))


# speedup vs baseline: 1.0196x; 1.0196x over previous
import functools

import jax
import jax.numpy as jnp
import numpy as np
from jax import lax
from jax.experimental import pallas as pl
from jax.experimental.pallas import tpu as pltpu
from jax.experimental.pallas import tpu_sc as plsc

NUM_EMB = 105220
D = 256
G = 128
L = 50
N = 6400
H = 128

NC, NS = 2, 16
NW = NC * NS
CHUNKS = (128, 72)
LANES = 16


def _sc_gather(table, idx, d, clamp_max=None):
    n_rows = idx.shape[0]
    rpw = n_rows // NW
    assert rpw == sum(CHUNKS)
    n_vecs = -(-rpw // LANES)
    pad = n_vecs * LANES
    mesh = plsc.VectorSubcoreMesh(core_axis_name="c", subcore_axis_name="s")

    @functools.partial(
        pl.kernel,
        out_type=jax.ShapeDtypeStruct((n_rows, d), jnp.float32),
        mesh=mesh,
        scratch_types=[
            pltpu.VMEM((pad,), jnp.int32),
            pltpu.VMEM((rpw, d), jnp.float32),
            pltpu.SemaphoreType.DMA,
            pltpu.SemaphoreType.DMA,
        ],
    )
    def gather_kernel(idx_hbm, table_hbm, out_hbm, idx_v, rows_v, sem_g, sem_w):
        wid = lax.axis_index("s") * NC + lax.axis_index("c")
        base = wid * rpw
        pltpu.sync_copy(idx_hbm.at[pl.ds(base, rpw)], idx_v.at[pl.ds(0, rpw)])
        if clamp_max is not None:
            hi = jnp.full((LANES,), clamp_max, jnp.int32)
            lo = jnp.zeros((LANES,), jnp.int32)
            for v in range(n_vecs):
                sl = pl.ds(v * LANES, LANES)
                idx_v[sl] = jnp.minimum(jnp.maximum(idx_v[sl], lo), hi)
        off = 0
        gathers = []
        for c in CHUNKS:
            gathers.append(
                pltpu.async_copy(
                    table_hbm.at[idx_v.at[pl.ds(off, c)]],
                    rows_v.at[pl.ds(off, c)],
                    sem_g,
                )
            )
            off += c
        writes = []
        off = 0
        for g, c in zip(gathers, CHUNKS):
            g.wait()
            writes.append(
                pltpu.async_copy(
                    rows_v.at[pl.ds(off, c)],
                    out_hbm.at[pl.ds(base + off, c)],
                    sem_w,
                )
            )
            off += c
        for w in writes:
            w.wait()

    return gather_kernel(idx, table)


NSTRIPE = 4
SROWS = G * L // NSTRIPE


def _project_body(e_hbm, w1_ref, b1_ref, w2_ref, b2_ref, batch_ref, x_ref,
                  e_v, *sems):
    copies = [
        pltpu.make_async_copy(
            e_hbm.at[pl.ds(k * SROWS, SROWS)],
            e_v.at[pl.ds(k * SROWS, SROWS)],
            sems[k],
        )
        for k in range(NSTRIPE)
    ]
    for c in copies:
        c.start()

    gps = G // NSTRIPE
    row = lax.broadcasted_iota(jnp.int32, (gps, SROWS), 0)
    col = lax.broadcasted_iota(jnp.int32, (gps, SROWS), 1)
    off = col - row * L
    seg = jnp.where((off >= 0) & (off < L), 1.0 / L, 0.0)

    hms = []
    for k in range(NSTRIPE):
        copies[k].wait()
        e = e_v[pl.ds(k * SROWS, SROWS), :]
        h = jnp.dot(e, w1_ref[...], preferred_element_type=jnp.float32)
        h = h + b1_ref[...]
        h = jnp.where(h >= 0, h, 0.01 * h)
        hms.append(jnp.dot(seg, h, preferred_element_type=jnp.float32))

    hm = jnp.concatenate(hms, axis=0)
    m = jnp.dot(hm, w2_ref[...], preferred_element_type=jnp.float32)
    m = m + b2_ref[...]
    out_copies = []
    for k in range(NSTRIPE):
        sl = pl.ds(k * SROWS, SROWS)
        b_row = jnp.broadcast_to(batch_ref[:, sl], (G, SROWS))
        g_col = lax.broadcasted_iota(jnp.int32, (G, SROWS), 0)
        oh_t = jnp.where(b_row == g_col, 1.0, 0.0)
        e_v[sl, :] = lax.dot_general(
            oh_t, m, (((0,), (0,)), ((), ())),
            preferred_element_type=jnp.float32)
        cp = pltpu.make_async_copy(e_v.at[sl], x_ref.at[sl], sems[k])
        cp.start()
        out_copies.append(cp)
    for cp in out_copies:
        cp.wait()


def _project(e, W1, b1, W2, b2, batch):
    return pl.pallas_call(
        _project_body,
        in_specs=[
            pl.BlockSpec(memory_space=pl.ANY),
            pl.BlockSpec(memory_space=pltpu.MemorySpace.VMEM),
            pl.BlockSpec(memory_space=pltpu.MemorySpace.VMEM),
            pl.BlockSpec(memory_space=pltpu.MemorySpace.VMEM),
            pl.BlockSpec(memory_space=pltpu.MemorySpace.VMEM),
            pl.BlockSpec(memory_space=pltpu.MemorySpace.VMEM),
        ],
        out_specs=pl.BlockSpec(memory_space=pl.ANY),
        out_shape=jax.ShapeDtypeStruct((N, D), jnp.float32),
        scratch_shapes=(
            [pltpu.VMEM((G * L, D), jnp.float32)]
            + [pltpu.SemaphoreType.DMA] * NSTRIPE
        ),
    )(e, W1, b1.reshape(1, H), W2, b2.reshape(1, D), batch.reshape(1, N))


def kernel(original_ids, batch, emb_table, W1, b1, W2, b2):
    ids = original_ids.astype(jnp.int32).reshape(-1)
    e = _sc_gather(emb_table, ids, D, clamp_max=NUM_EMB - 1)
    return _project(e, W1, b1, W2, b2, batch.astype(jnp.int32))
